# Initial kernel scaffold; baseline (speedup 1.0000x reference)
import jax
import jax.numpy as jnp
from jax.experimental import pallas as pl


def kernel(hidden_states, gate_w, w_gate, w_up, w_down, shared_gate, shared_up, shared_down):
    raise NotImplementedError("write your pallas kernel here")


# speedup vs baseline: 1.2661x; 1.2661x over previous
import functools

import jax
import jax.numpy as jnp
from jax.experimental import pallas as pl
from jax.experimental.pallas import tpu as pltpu

_T = 2048
_D = 1024
_E = 8
_F = 512
_SF = 2
_LIMIT = 7.0
_RSF = 2.5
_BT = 256


def _moe_kernel(x_ref, gw_ref, wg_ref, wu_ref, wd_ref,
                sg_ref, su_ref, sd_ref, out_ref, comb_ref):
    e = pl.program_id(0)
    t = pl.program_id(1)
    row = pl.ds(t * _BT, _BT)
    x = x_ref[row, :]
    xb = x.astype(jnp.bfloat16)

    @pl.when(e == 0)
    def _router_and_shared():
        logits = jnp.dot(x, gw_ref[...].T, preferred_element_type=jnp.float32)
        scores = jnp.sqrt(jax.nn.softplus(logits))
        col = jax.lax.broadcasted_iota(jnp.int32, scores.shape, 1)
        m1 = jnp.max(scores, axis=1, keepdims=True)
        i1 = jnp.min(jnp.where(scores == m1, col, _E), axis=1, keepdims=True)
        masked = jnp.where(col == i1, -jnp.inf, scores)
        m2 = jnp.max(masked, axis=1, keepdims=True)
        i2 = jnp.min(jnp.where(masked == m2, col, _E), axis=1, keepdims=True)
        s = m1 + m2
        comb = (jnp.where(col == i1, m1, 0.0) +
                jnp.where(col == i2, m2, 0.0)) * (_RSF / s)
        comb_ref[row, :] = comb

        sgb = sg_ref[...].astype(jnp.bfloat16)
        sub = su_ref[...].astype(jnp.bfloat16)
        sdb = sd_ref[...].astype(jnp.bfloat16)
        dn = (((1,), (1,)), ((), ()))
        a = jax.lax.dot_general(xb, sgb, dn, preferred_element_type=jnp.float32)
        b = jax.lax.dot_general(xb, sub, dn, preferred_element_type=jnp.float32)
        hs = (a * jax.nn.sigmoid(a) * b).astype(jnp.bfloat16)
        shared = jax.lax.dot_general(hs, sdb, dn, preferred_element_type=jnp.float32)
        out_ref[row, :] = shared

    wg = wg_ref[0].astype(jnp.bfloat16)
    wu = wu_ref[0].astype(jnp.bfloat16)
    wd = wd_ref[0].astype(jnp.bfloat16)
    dn = (((1,), (1,)), ((), ()))
    g = jax.lax.dot_general(xb, wg, dn, preferred_element_type=jnp.float32)
    u = jax.lax.dot_general(xb, wu, dn, preferred_element_type=jnp.float32)
    g = jnp.minimum(g, _LIMIT)
    u = jnp.clip(u, -_LIMIT, _LIMIT)
    h = ((g * jax.nn.sigmoid(g)) * u).astype(jnp.bfloat16)
    y = jax.lax.dot_general(h, wd, dn, preferred_element_type=jnp.float32)

    comb = comb_ref[row, :]
    col = jax.lax.broadcasted_iota(jnp.int32, comb.shape, 1)
    we = jnp.sum(jnp.where(col == e, comb, 0.0), axis=1, keepdims=True)
    out_ref[row, :] += y * we


def kernel(hidden_states, gate_w, w_gate, w_up, w_down,
           shared_gate, shared_up, shared_down):
    org_shape = hidden_states.shape
    x = hidden_states.reshape(-1, org_shape[-1])

    nt = _T // _BT
    out = pl.pallas_call(
        _moe_kernel,
        grid=(_E, nt),
        in_specs=[
            pl.BlockSpec((_T, _D), lambda e, t: (0, 0)),
            pl.BlockSpec((_E, _D), lambda e, t: (0, 0)),
            pl.BlockSpec((1, _F, _D), lambda e, t: (e, 0, 0)),
            pl.BlockSpec((1, _F, _D), lambda e, t: (e, 0, 0)),
            pl.BlockSpec((1, _D, _F), lambda e, t: (e, 0, 0)),
            pl.BlockSpec((_F * _SF, _D), lambda e, t: (0, 0)),
            pl.BlockSpec((_F * _SF, _D), lambda e, t: (0, 0)),
            pl.BlockSpec((_D, _F * _SF), lambda e, t: (0, 0)),
        ],
        out_specs=pl.BlockSpec((_T, _D), lambda e, t: (0, 0)),
        out_shape=jax.ShapeDtypeStruct((_T, _D), jnp.float32),
        scratch_shapes=[pltpu.VMEM((_T, _E), jnp.float32)],
    )(x, gate_w, w_gate, w_up, w_down, shared_gate, shared_up, shared_down)

    return out.reshape(org_shape)
